# Initial kernel scaffold; baseline (speedup 1.0000x reference)
import jax
import jax.numpy as jnp
from jax.experimental import pallas as pl


def kernel(features, edge_index, W_self0, W_neigh0, b0, W_self1, W_neigh1, b1):
    raise NotImplementedError("write your pallas kernel here")


# speedup vs baseline: 4.7224x; 4.7224x over previous
import functools

import jax
import jax.numpy as jnp
from jax import lax
from jax.experimental import pallas as pl
from jax.experimental.pallas import tpu as pltpu
from jax.experimental.pallas import tpu_sc as plsc

N = 10000
D = 128
E = 320000

NC = 2
NS = 16
NW = NC * NS
CHUNK = 64
CPW = 160
IB = 32
NOUTER = CPW // IB
E_PAD = NW * CPW * CHUNK
ROWS_PAD = 10112
RPS = ROWS_PAD // NS

_mesh = plsc.VectorSubcoreMesh(core_axis_name="c", subcore_axis_name="s")


def _sc_agg_body(x_hbm, src_hbm, dst_hbm, z_hbm, agg_out,
                 spmem_agg, ring_s, ring_d, rows, gsem, ssem):
    c = lax.axis_index("c")
    s = lax.axis_index("s")
    w = c * NS + s
    base = w * CPW

    def g_copy(om, j, b):
        return pltpu.make_async_copy(
            x_hbm.at[ring_s.at[om, j]], rows.at[b], gsem)

    def s_copy(om, j, b):
        return pltpu.make_async_copy(
            rows.at[b], spmem_agg.at[ring_d.at[om, j]], ssem)

    pltpu.sync_copy(z_hbm, spmem_agg.at[pl.ds(s * RPS, RPS)])
    pltpu.sync_copy(src_hbm.at[pl.ds(base, IB)], ring_s.at[0])
    pltpu.sync_copy(dst_hbm.at[pl.ds(base, IB)], ring_d.at[0])
    plsc.subcore_barrier()

    g_copy(0, 0, 0).start()

    def outer(o, carry):
        om = lax.rem(o, 2)

        def inner(j, carry2):
            i = o * IB + j
            b = lax.rem(j, 2)
            g_copy(om, j, b).wait()
            pltpu.async_copy(rows.at[b], spmem_agg.at[ring_d.at[om, j]],
                             ssem, add=True)

            @pl.when(i >= 1)
            def _():
                s_copy(om, j, b).wait()

            @pl.when(j < IB - 1)
            def _():
                g_copy(om, j + 1, 1 - b).start()

            return carry2

        lax.fori_loop(0, IB, inner, 0)

        @pl.when(o < NOUTER - 1)
        def _():
            om1 = lax.rem(o + 1, 2)
            pltpu.sync_copy(src_hbm.at[pl.ds(base + (o + 1) * IB, IB)],
                            ring_s.at[om1])
            pltpu.sync_copy(dst_hbm.at[pl.ds(base + (o + 1) * IB, IB)],
                            ring_d.at[om1])
            g_copy(om1, 0, 0).start()

        return carry

    lax.fori_loop(0, NOUTER, outer, 0)
    s_copy((NOUTER - 1) % 2, IB - 1, (IB - 1) % 2).wait()
    plsc.subcore_barrier()

    pltpu.sync_copy(spmem_agg.at[pl.ds(s * RPS, RPS)],
                    agg_out.at[c, pl.ds(s * RPS, RPS), :])


_sc_agg = pl.kernel(
    _sc_agg_body,
    mesh=_mesh,
    out_type=jax.ShapeDtypeStruct((NC, ROWS_PAD, D), jnp.float32),
    scratch_types=[
        pltpu.VMEM_SHARED((ROWS_PAD, D), jnp.float32),
        pltpu.VMEM((2, IB, CHUNK), jnp.int32),
        pltpu.VMEM((2, IB, CHUNK), jnp.int32),
        pltpu.VMEM((2, CHUNK, D), jnp.float32),
        pltpu.SemaphoreType.DMA,
        pltpu.SemaphoreType.DMA,
    ],
)


_TC_R = 1000


def _tc_layer_body(relu, h_ref, p0_ref, p1_ref, dinv_ref,
                   ws_ref, wn_ref, b_ref, o_ref):
    hn = (p0_ref[0] + p1_ref[0]) * dinv_ref[...]
    acc = jnp.dot(h_ref[...], ws_ref[...], preferred_element_type=jnp.float32)
    acc = acc + jnp.dot(hn, wn_ref[...], preferred_element_type=jnp.float32)
    acc = acc + b_ref[...]
    o_ref[...] = jnp.maximum(acc, 0.0) if relu else acc


def _tc_layer(relu, h, agg, dinv, Ws, Wn, b):
    return pl.pallas_call(
        functools.partial(_tc_layer_body, relu),
        grid=(N // _TC_R,),
        in_specs=[
            pl.BlockSpec((_TC_R, D), lambda i: (i, 0)),
            pl.BlockSpec((1, _TC_R, D), lambda i: (0, i, 0)),
            pl.BlockSpec((1, _TC_R, D), lambda i: (1, i, 0)),
            pl.BlockSpec((_TC_R, 1), lambda i: (i, 0)),
            pl.BlockSpec((D, D), lambda i: (0, 0)),
            pl.BlockSpec((D, D), lambda i: (0, 0)),
            pl.BlockSpec((1, D), lambda i: (0, 0)),
        ],
        out_specs=pl.BlockSpec((_TC_R, D), lambda i: (i, 0)),
        out_shape=jax.ShapeDtypeStruct((N, D), jnp.float32),
    )(h, agg, agg, dinv, Ws, Wn, b)


def kernel(features, edge_index, W_self0, W_neigh0, b0, W_self1, W_neigh1, b1):
    src = edge_index[0]
    dst = edge_index[1]
    pad = E_PAD - E
    ar = jnp.arange(pad, dtype=jnp.int32)
    src_p = jnp.concatenate([src, (ar * 37) % N]).reshape(NW * CPW, CHUNK)
    dst_p = jnp.concatenate([dst, N + ar % (ROWS_PAD - N)]).reshape(
        NW * CPW, CHUNK)
    z128 = jnp.zeros((RPS, D), jnp.float32)

    deg = jnp.zeros((N,), jnp.float32).at[dst].add(1.0)
    dinv = (1.0 / jnp.maximum(deg, 1.0)).reshape(N, 1)

    aggA = _sc_agg(features, src_p, dst_p, z128)
    h1 = _tc_layer(True, features, aggA, dinv, W_self0, W_neigh0,
                   b0.reshape(1, D))
    aggB = _sc_agg(h1, src_p, dst_p, z128)
    out = _tc_layer(False, h1, aggB, dinv, W_self1, W_neigh1,
                    b1.reshape(1, D))
    return out


# speedup vs baseline: 10.2908x; 2.1791x over previous
import functools

import jax
import jax.numpy as jnp
from jax import lax
from jax.experimental import pallas as pl
from jax.experimental.pallas import tpu as pltpu
from jax.experimental.pallas import tpu_sc as plsc

N = 10000
D = 128
E = 320000

NC = 2
NS = 16
NW = NC * NS
CHUNK = 64
CPW = 160
IB = 32
NOUTER = CPW // IB
E_PAD = NW * CPW * CHUNK
ROWS_PAD = 10112
RPS = ROWS_PAD // NS

_mesh = plsc.VectorSubcoreMesh(core_axis_name="c", subcore_axis_name="s")


def _sc_agg_body(nbuf, x_hbm, src_hbm, dst_hbm, z_hbm, agg_out,
                 spmem_agg, ring_s, ring_d, rows, gsem, ssem):
    c = lax.axis_index("c")
    s = lax.axis_index("s")
    w = c * NS + s
    base = w * CPW

    def g_copy(om, j, b):
        return pltpu.make_async_copy(
            x_hbm.at[ring_s.at[om, j]], rows.at[b], gsem)

    def s_copy(om, j, b):
        return pltpu.make_async_copy(
            rows.at[b], spmem_agg.at[ring_d.at[om, j]], ssem)

    pltpu.sync_copy(z_hbm, spmem_agg.at[pl.ds(s * RPS, RPS)])
    pltpu.sync_copy(src_hbm.at[pl.ds(base, IB)], ring_s.at[0])
    pltpu.sync_copy(dst_hbm.at[pl.ds(base, IB)], ring_d.at[0])
    plsc.subcore_barrier()

    for t in range(nbuf - 1):
        g_copy(0, t, t).start()

    def outer(o, carry):
        om = lax.rem(o, 2)

        def inner(j, carry2):
            i = o * IB + j
            b = lax.rem(i, nbuf)
            g_copy(om, j, b).wait()
            pltpu.async_copy(rows.at[b], spmem_agg.at[ring_d.at[om, j]],
                             ssem, add=True)

            @pl.when(j >= 1)
            def _():
                s_copy(om, j, b).wait()

            @pl.when(j < IB - (nbuf - 1))
            def _():
                g_copy(om, j + nbuf - 1,
                       lax.rem(i + nbuf - 1, nbuf)).start()

            return carry2

        lax.fori_loop(0, IB, inner, 0)

        @pl.when(o < NOUTER - 1)
        def _():
            s_copy(om, IB - 1, lax.rem(o * IB + IB - 1, nbuf)).wait()
            om1 = lax.rem(o + 1, 2)
            pltpu.sync_copy(src_hbm.at[pl.ds(base + (o + 1) * IB, IB)],
                            ring_s.at[om1])
            pltpu.sync_copy(dst_hbm.at[pl.ds(base + (o + 1) * IB, IB)],
                            ring_d.at[om1])
            for t in range(nbuf - 1):
                g_copy(om1, t, lax.rem((o + 1) * IB + t, nbuf)).start()

        return carry

    lax.fori_loop(0, NOUTER, outer, 0)
    s_copy((NOUTER - 1) % 2, IB - 1, (CPW - 1) % nbuf).wait()
    plsc.subcore_barrier()

    pltpu.sync_copy(spmem_agg.at[pl.ds(s * RPS, RPS)],
                    agg_out.at[c, pl.ds(s * RPS, RPS), :])


_sc_agg = pl.kernel(
    functools.partial(_sc_agg_body, 3),
    mesh=_mesh,
    out_type=jax.ShapeDtypeStruct((NC, ROWS_PAD, D), jnp.float32),
    scratch_types=[
        pltpu.VMEM_SHARED((ROWS_PAD, D), jnp.float32),
        pltpu.VMEM((2, IB, CHUNK), jnp.int32),
        pltpu.VMEM((2, IB, CHUNK), jnp.int32),
        pltpu.VMEM((3, CHUNK, D), jnp.float32),
        pltpu.SemaphoreType.DMA,
        pltpu.SemaphoreType.DMA,
    ],
)


CHUNK_D = 128
CPW_D = E_PAD // (NW * CHUNK_D)
IB_D = 16
NOUTER_D = CPW_D // IB_D


def _sc_deg_body(dst_hbm, z_hbm, ones_hbm, deg_out,
                 spmem_deg, ring_d, ones_v, ssem):
    c = lax.axis_index("c")
    s = lax.axis_index("s")
    w = c * NS + s
    base = w * CPW_D

    def s_copy(om, j):
        return pltpu.make_async_copy(
            ones_v, spmem_deg.at[ring_d.at[om, j]], ssem)

    pltpu.sync_copy(z_hbm, spmem_deg.at[pl.ds(s * RPS, RPS)])
    pltpu.sync_copy(ones_hbm, ones_v)
    pltpu.sync_copy(dst_hbm.at[pl.ds(base, IB_D)], ring_d.at[0])
    plsc.subcore_barrier()

    def outer(o, carry):
        om = lax.rem(o, 2)

        def inner(j, carry2):
            i = o * IB_D + j
            pltpu.async_copy(ones_v, spmem_deg.at[ring_d.at[om, j]],
                             ssem, add=True)

            @pl.when(i >= 2)
            def _():
                s_copy(om, j).wait()

            return carry2

        lax.fori_loop(0, IB_D, inner, 0)

        @pl.when(o < NOUTER_D - 1)
        def _():
            pltpu.sync_copy(
                dst_hbm.at[pl.ds(base + (o + 1) * IB_D, IB_D)],
                ring_d.at[lax.rem(o + 1, 2)])

        return carry

    lax.fori_loop(0, NOUTER_D, outer, 0)
    s_copy(0, 0).wait()
    s_copy(0, 0).wait()
    plsc.subcore_barrier()

    pltpu.sync_copy(spmem_deg.at[pl.ds(s * RPS, RPS)],
                    deg_out.at[c, pl.ds(s * RPS, RPS), :])


_sc_deg = pl.kernel(
    _sc_deg_body,
    mesh=_mesh,
    out_type=jax.ShapeDtypeStruct((NC, ROWS_PAD, D), jnp.float32),
    scratch_types=[
        pltpu.VMEM_SHARED((ROWS_PAD, D), jnp.float32),
        pltpu.VMEM((2, IB_D, CHUNK_D), jnp.int32),
        pltpu.VMEM((CHUNK_D, D), jnp.float32),
        pltpu.SemaphoreType.DMA,
    ],
)


_TC_R = 1000


def _tc_layer_body(relu, h_ref, p0_ref, p1_ref, d0_ref, d1_ref,
                   ws_ref, wn_ref, b_ref, o_ref):
    deg = d0_ref[0][:, 0:1] + d1_ref[0][:, 0:1]
    hn = (p0_ref[0] + p1_ref[0]) * (1.0 / jnp.maximum(deg, 1.0))
    acc = jnp.dot(h_ref[...], ws_ref[...], preferred_element_type=jnp.float32)
    acc = acc + jnp.dot(hn, wn_ref[...], preferred_element_type=jnp.float32)
    acc = acc + b_ref[...]
    o_ref[...] = jnp.maximum(acc, 0.0) if relu else acc


def _tc_layer(relu, h, agg, degp, Ws, Wn, b):
    return pl.pallas_call(
        functools.partial(_tc_layer_body, relu),
        grid=(N // _TC_R,),
        in_specs=[
            pl.BlockSpec((_TC_R, D), lambda i: (i, 0)),
            pl.BlockSpec((1, _TC_R, D), lambda i: (0, i, 0)),
            pl.BlockSpec((1, _TC_R, D), lambda i: (1, i, 0)),
            pl.BlockSpec((1, _TC_R, D), lambda i: (0, i, 0)),
            pl.BlockSpec((1, _TC_R, D), lambda i: (1, i, 0)),
            pl.BlockSpec((D, D), lambda i: (0, 0)),
            pl.BlockSpec((D, D), lambda i: (0, 0)),
            pl.BlockSpec((1, D), lambda i: (0, 0)),
        ],
        out_specs=pl.BlockSpec((_TC_R, D), lambda i: (i, 0)),
        out_shape=jax.ShapeDtypeStruct((N, D), jnp.float32),
    )(h, agg, agg, degp, degp, Ws, Wn, b)


def kernel(features, edge_index, W_self0, W_neigh0, b0, W_self1, W_neigh1, b1):
    src = edge_index[0]
    dst = edge_index[1]
    pad = E_PAD - E
    ar = jnp.arange(pad, dtype=jnp.int32)
    src_p = jnp.concatenate([src, (ar * 37) % N]).reshape(NW * CPW, CHUNK)
    dst_p = jnp.concatenate([dst, N + ar % (ROWS_PAD - N)]).reshape(
        NW * CPW, CHUNK)
    z128 = jnp.zeros((RPS, D), jnp.float32)
    ones128 = jnp.ones((CHUNK_D, D), jnp.float32)
    dst2_p = dst_p.reshape(NW * CPW_D, CHUNK_D)

    degp = _sc_deg(dst2_p, z128, ones128)
    aggA = _sc_agg(features, src_p, dst_p, z128)
    h1 = _tc_layer(True, features, aggA, degp, W_self0, W_neigh0,
                   b0.reshape(1, D))
    aggB = _sc_agg(h1, src_p, dst_p, z128)
    out = _tc_layer(False, h1, aggB, degp, W_self1, W_neigh1,
                    b1.reshape(1, D))
    return out


# speedup vs baseline: 10.5470x; 1.0249x over previous
import functools

import jax
import jax.numpy as jnp
from jax import lax
from jax.experimental import pallas as pl
from jax.experimental.pallas import tpu as pltpu
from jax.experimental.pallas import tpu_sc as plsc

N = 10000
D = 128
E = 320000

NC = 2
NS = 16
NW = NC * NS
CHUNK = 64
CPW = 160
IB = 16
NOUTER = CPW // IB
E_PAD = NW * CPW * CHUNK
ROWS_PAD = 10112
RPS = ROWS_PAD // NS

_mesh = plsc.VectorSubcoreMesh(core_axis_name="c", subcore_axis_name="s")


def _sc_agg_body(nbuf, x_hbm, src_hbm, dst_hbm, z_hbm, agg_out,
                 spmem_agg, ring_s, ring_d, rows, gsem, ssem):
    c = lax.axis_index("c")
    s = lax.axis_index("s")
    w = c * NS + s
    base = w * CPW

    def g_copy(om, j, b):
        return pltpu.make_async_copy(
            x_hbm.at[ring_s.at[om, j]], rows.at[b], gsem)

    def s_copy(om, j, b):
        return pltpu.make_async_copy(
            rows.at[b], spmem_agg.at[ring_d.at[om, j]], ssem)

    pltpu.sync_copy(z_hbm, spmem_agg.at[pl.ds(s * RPS, RPS)])
    pltpu.sync_copy(src_hbm.at[pl.ds(base, IB)], ring_s.at[0])
    pltpu.sync_copy(dst_hbm.at[pl.ds(base, IB)], ring_d.at[0])
    plsc.subcore_barrier()

    for t in range(nbuf - 1):
        g_copy(0, t, t).start()

    def outer(o, carry):
        om = lax.rem(o, 2)

        def inner(j, carry2):
            i = o * IB + j
            b = lax.rem(i, nbuf)
            g_copy(om, j, b).wait()
            pltpu.async_copy(rows.at[b], spmem_agg.at[ring_d.at[om, j]],
                             ssem, add=True)

            @pl.when(j >= 1)
            def _():
                s_copy(om, j, b).wait()

            @pl.when(j < IB - (nbuf - 1))
            def _():
                g_copy(om, j + nbuf - 1,
                       lax.rem(i + nbuf - 1, nbuf)).start()

            return carry2

        lax.fori_loop(0, IB, inner, 0)

        @pl.when(o < NOUTER - 1)
        def _():
            s_copy(om, IB - 1, lax.rem(o * IB + IB - 1, nbuf)).wait()
            om1 = lax.rem(o + 1, 2)
            pltpu.sync_copy(src_hbm.at[pl.ds(base + (o + 1) * IB, IB)],
                            ring_s.at[om1])
            pltpu.sync_copy(dst_hbm.at[pl.ds(base + (o + 1) * IB, IB)],
                            ring_d.at[om1])
            for t in range(nbuf - 1):
                g_copy(om1, t, lax.rem((o + 1) * IB + t, nbuf)).start()

        return carry

    lax.fori_loop(0, NOUTER, outer, 0)
    s_copy((NOUTER - 1) % 2, IB - 1, (CPW - 1) % nbuf).wait()
    plsc.subcore_barrier()

    pltpu.sync_copy(spmem_agg.at[pl.ds(s * RPS, RPS)],
                    agg_out.at[c, pl.ds(s * RPS, RPS), :])


_sc_agg = pl.kernel(
    functools.partial(_sc_agg_body, 4),
    mesh=_mesh,
    out_type=jax.ShapeDtypeStruct((NC, ROWS_PAD, D), jnp.float32),
    scratch_types=[
        pltpu.VMEM_SHARED((ROWS_PAD, D), jnp.float32),
        pltpu.VMEM((2, IB, CHUNK), jnp.int32),
        pltpu.VMEM((2, IB, CHUNK), jnp.int32),
        pltpu.VMEM((4, CHUNK, D), jnp.float32),
        pltpu.SemaphoreType.DMA,
        pltpu.SemaphoreType.DMA,
    ],
)


CHUNK_D = 128
CPW_D = E_PAD // (NW * CHUNK_D)
IB_D = 16
NOUTER_D = CPW_D // IB_D


def _sc_deg_body(dst_hbm, z_hbm, ones_hbm, deg_out,
                 spmem_deg, ring_d, ones_v, ssem):
    c = lax.axis_index("c")
    s = lax.axis_index("s")
    w = c * NS + s
    base = w * CPW_D

    def s_copy(om, j):
        return pltpu.make_async_copy(
            ones_v, spmem_deg.at[ring_d.at[om, j]], ssem)

    pltpu.sync_copy(z_hbm, spmem_deg.at[pl.ds(s * RPS, RPS)])
    pltpu.sync_copy(ones_hbm, ones_v)
    pltpu.sync_copy(dst_hbm.at[pl.ds(base, IB_D)], ring_d.at[0])
    plsc.subcore_barrier()

    def outer(o, carry):
        om = lax.rem(o, 2)

        def inner(j, carry2):
            i = o * IB_D + j
            pltpu.async_copy(ones_v, spmem_deg.at[ring_d.at[om, j]],
                             ssem, add=True)

            @pl.when(i >= 2)
            def _():
                s_copy(om, j).wait()

            return carry2

        lax.fori_loop(0, IB_D, inner, 0)

        @pl.when(o < NOUTER_D - 1)
        def _():
            pltpu.sync_copy(
                dst_hbm.at[pl.ds(base + (o + 1) * IB_D, IB_D)],
                ring_d.at[lax.rem(o + 1, 2)])

        return carry

    lax.fori_loop(0, NOUTER_D, outer, 0)
    s_copy(0, 0).wait()
    s_copy(0, 0).wait()
    plsc.subcore_barrier()

    pltpu.sync_copy(spmem_deg.at[pl.ds(s * RPS, RPS)],
                    deg_out.at[c, pl.ds(s * RPS, RPS), :])


_sc_deg = pl.kernel(
    _sc_deg_body,
    mesh=_mesh,
    out_type=jax.ShapeDtypeStruct((NC, ROWS_PAD, D), jnp.float32),
    scratch_types=[
        pltpu.VMEM_SHARED((ROWS_PAD, D), jnp.float32),
        pltpu.VMEM((2, IB_D, CHUNK_D), jnp.int32),
        pltpu.VMEM((CHUNK_D, D), jnp.float32),
        pltpu.SemaphoreType.DMA,
    ],
)


_TC_R = 1000


def _tc_layer_body(relu, h_ref, p0_ref, p1_ref, d0_ref, d1_ref,
                   ws_ref, wn_ref, b_ref, o_ref):
    deg = d0_ref[0][:, 0:1] + d1_ref[0][:, 0:1]
    hn = (p0_ref[0] + p1_ref[0]) * (1.0 / jnp.maximum(deg, 1.0))
    acc = jnp.dot(h_ref[...], ws_ref[...], preferred_element_type=jnp.float32)
    acc = acc + jnp.dot(hn, wn_ref[...], preferred_element_type=jnp.float32)
    acc = acc + b_ref[...]
    o_ref[...] = jnp.maximum(acc, 0.0) if relu else acc


def _tc_layer(relu, h, agg, degp, Ws, Wn, b):
    return pl.pallas_call(
        functools.partial(_tc_layer_body, relu),
        grid=(N // _TC_R,),
        in_specs=[
            pl.BlockSpec((_TC_R, D), lambda i: (i, 0)),
            pl.BlockSpec((1, _TC_R, D), lambda i: (0, i, 0)),
            pl.BlockSpec((1, _TC_R, D), lambda i: (1, i, 0)),
            pl.BlockSpec((1, _TC_R, D), lambda i: (0, i, 0)),
            pl.BlockSpec((1, _TC_R, D), lambda i: (1, i, 0)),
            pl.BlockSpec((D, D), lambda i: (0, 0)),
            pl.BlockSpec((D, D), lambda i: (0, 0)),
            pl.BlockSpec((1, D), lambda i: (0, 0)),
        ],
        out_specs=pl.BlockSpec((_TC_R, D), lambda i: (i, 0)),
        out_shape=jax.ShapeDtypeStruct((N, D), jnp.float32),
    )(h, agg, agg, degp, degp, Ws, Wn, b)


def kernel(features, edge_index, W_self0, W_neigh0, b0, W_self1, W_neigh1, b1):
    src = edge_index[0]
    dst = edge_index[1]
    pad = E_PAD - E
    ar = jnp.arange(pad, dtype=jnp.int32)
    src_p = jnp.concatenate([src, (ar * 37) % N]).reshape(NW * CPW, CHUNK)
    dst_p = jnp.concatenate([dst, N + ar % (ROWS_PAD - N)]).reshape(
        NW * CPW, CHUNK)
    z128 = jnp.zeros((RPS, D), jnp.float32)
    ones128 = jnp.ones((CHUNK_D, D), jnp.float32)
    dst2_p = dst_p.reshape(NW * CPW_D, CHUNK_D)

    degp = _sc_deg(dst2_p, z128, ones128)
    aggA = _sc_agg(features, src_p, dst_p, z128)
    h1 = _tc_layer(True, features, aggA, degp, W_self0, W_neigh0,
                   b0.reshape(1, D))
    aggB = _sc_agg(h1, src_p, dst_p, z128)
    out = _tc_layer(False, h1, aggB, degp, W_self1, W_neigh1,
                    b1.reshape(1, D))
    return out


# speedup vs baseline: 10.5472x; 1.0000x over previous
import functools

import jax
import jax.numpy as jnp
from jax import lax
from jax.experimental import pallas as pl
from jax.experimental.pallas import tpu as pltpu
from jax.experimental.pallas import tpu_sc as plsc

N = 10000
D = 128
E = 320000

NC = 2
NS = 16
NW = NC * NS
CHUNK = 64
CPW = 160
IB = 16
NOUTER = CPW // IB
E_PAD = NW * CPW * CHUNK
ROWS_PAD = 10112
RPS = ROWS_PAD // NS

_mesh = plsc.VectorSubcoreMesh(core_axis_name="c", subcore_axis_name="s")


def _sc_agg_body(nbuf, x_hbm, src_hbm, dst_hbm, z_hbm, agg_out,
                 spmem_agg, ring_s, ring_d, rows, gsem, ssem):
    c = lax.axis_index("c")
    s = lax.axis_index("s")
    w = c * NS + s
    base = w * CPW

    def g_copy(om, j, b):
        return pltpu.make_async_copy(
            x_hbm.at[ring_s.at[om, j]], rows.at[b], gsem)

    def s_copy(om, j, b):
        return pltpu.make_async_copy(
            rows.at[b], spmem_agg.at[ring_d.at[om, j]], ssem)

    pltpu.sync_copy(z_hbm, spmem_agg.at[pl.ds(s * RPS, RPS)])
    pltpu.sync_copy(src_hbm.at[pl.ds(base, IB)], ring_s.at[0])
    pltpu.sync_copy(dst_hbm.at[pl.ds(base, IB)], ring_d.at[0])
    plsc.subcore_barrier()

    for t in range(nbuf - 1):
        g_copy(0, t, t).start()

    def outer(o, carry):
        om = lax.rem(o, 2)

        def inner(j, carry2):
            i = o * IB + j
            b = lax.rem(i, nbuf)
            g_copy(om, j, b).wait()
            pltpu.async_copy(rows.at[b], spmem_agg.at[ring_d.at[om, j]],
                             ssem, add=True)

            @pl.when(j >= 1)
            def _():
                s_copy(om, j, b).wait()

            @pl.when(j < IB - (nbuf - 1))
            def _():
                g_copy(om, j + nbuf - 1,
                       lax.rem(i + nbuf - 1, nbuf)).start()

            return carry2

        lax.fori_loop(0, IB, inner, 0)

        @pl.when(o < NOUTER - 1)
        def _():
            s_copy(om, IB - 1, lax.rem(o * IB + IB - 1, nbuf)).wait()
            om1 = lax.rem(o + 1, 2)
            pltpu.sync_copy(src_hbm.at[pl.ds(base + (o + 1) * IB, IB)],
                            ring_s.at[om1])
            pltpu.sync_copy(dst_hbm.at[pl.ds(base + (o + 1) * IB, IB)],
                            ring_d.at[om1])
            for t in range(nbuf - 1):
                g_copy(om1, t, lax.rem((o + 1) * IB + t, nbuf)).start()

        return carry

    lax.fori_loop(0, NOUTER, outer, 0)
    s_copy((NOUTER - 1) % 2, IB - 1, (CPW - 1) % nbuf).wait()
    plsc.subcore_barrier()

    pltpu.sync_copy(spmem_agg.at[pl.ds(s * RPS, RPS)],
                    agg_out.at[c, pl.ds(s * RPS, RPS), :])


_sc_agg = pl.kernel(
    functools.partial(_sc_agg_body, 4),
    mesh=_mesh,
    out_type=jax.ShapeDtypeStruct((NC, ROWS_PAD, D), jnp.float32),
    scratch_types=[
        pltpu.VMEM_SHARED((ROWS_PAD, D), jnp.float32),
        pltpu.VMEM((2, IB, CHUNK), jnp.int32),
        pltpu.VMEM((2, IB, CHUNK), jnp.int32),
        pltpu.VMEM((4, CHUNK, D), jnp.float32),
        pltpu.SemaphoreType.DMA,
        pltpu.SemaphoreType.DMA,
    ],
)


CHUNK_D = 128
CPW_D = E_PAD // (NW * CHUNK_D)
IB_D = 16
NOUTER_D = CPW_D // IB_D


def _sc_deg_body(dst_hbm, z_hbm, ones_hbm, deg_out,
                 spmem_deg, ring_d, ones_v, ssem):
    c = lax.axis_index("c")
    s = lax.axis_index("s")
    w = c * NS + s
    base = w * CPW_D

    def s_copy(om, j):
        return pltpu.make_async_copy(
            ones_v, spmem_deg.at[ring_d.at[om, j]], ssem)

    pltpu.sync_copy(z_hbm, spmem_deg.at[pl.ds(s * RPS, RPS)])
    pltpu.sync_copy(ones_hbm, ones_v)
    pltpu.sync_copy(dst_hbm.at[pl.ds(base, IB_D)], ring_d.at[0])
    plsc.subcore_barrier()

    def outer(o, carry):
        om = lax.rem(o, 2)

        def inner(j, carry2):
            i = o * IB_D + j
            pltpu.async_copy(ones_v, spmem_deg.at[ring_d.at[om, j]],
                             ssem, add=True)

            @pl.when(i >= 4)
            def _():
                s_copy(om, j).wait()

            return carry2

        lax.fori_loop(0, IB_D, inner, 0)

        @pl.when(o < NOUTER_D - 1)
        def _():
            pltpu.sync_copy(
                dst_hbm.at[pl.ds(base + (o + 1) * IB_D, IB_D)],
                ring_d.at[lax.rem(o + 1, 2)])

        return carry

    lax.fori_loop(0, NOUTER_D, outer, 0)
    for _ in range(4):
        s_copy(0, 0).wait()
    plsc.subcore_barrier()

    pltpu.sync_copy(spmem_deg.at[pl.ds(s * RPS, RPS)],
                    deg_out.at[c, pl.ds(s * RPS, RPS), :])


_sc_deg = pl.kernel(
    _sc_deg_body,
    mesh=_mesh,
    out_type=jax.ShapeDtypeStruct((NC, ROWS_PAD, D), jnp.float32),
    scratch_types=[
        pltpu.VMEM_SHARED((ROWS_PAD, D), jnp.float32),
        pltpu.VMEM((2, IB_D, CHUNK_D), jnp.int32),
        pltpu.VMEM((CHUNK_D, D), jnp.float32),
        pltpu.SemaphoreType.DMA,
    ],
)


_TC_R = 1000


def _tc_layer_body(relu, h_ref, p0_ref, p1_ref, d0_ref, d1_ref,
                   ws_ref, wn_ref, b_ref, o_ref):
    deg = d0_ref[0][:, 0:1] + d1_ref[0][:, 0:1]
    hn = (p0_ref[0] + p1_ref[0]) * (1.0 / jnp.maximum(deg, 1.0))
    acc = jnp.dot(h_ref[...], ws_ref[...], preferred_element_type=jnp.float32)
    acc = acc + jnp.dot(hn, wn_ref[...], preferred_element_type=jnp.float32)
    acc = acc + b_ref[...]
    o_ref[...] = jnp.maximum(acc, 0.0) if relu else acc


def _tc_layer(relu, h, agg, degp, Ws, Wn, b):
    return pl.pallas_call(
        functools.partial(_tc_layer_body, relu),
        grid=(N // _TC_R,),
        in_specs=[
            pl.BlockSpec((_TC_R, D), lambda i: (i, 0)),
            pl.BlockSpec((1, _TC_R, D), lambda i: (0, i, 0)),
            pl.BlockSpec((1, _TC_R, D), lambda i: (1, i, 0)),
            pl.BlockSpec((1, _TC_R, D), lambda i: (0, i, 0)),
            pl.BlockSpec((1, _TC_R, D), lambda i: (1, i, 0)),
            pl.BlockSpec((D, D), lambda i: (0, 0)),
            pl.BlockSpec((D, D), lambda i: (0, 0)),
            pl.BlockSpec((1, D), lambda i: (0, 0)),
        ],
        out_specs=pl.BlockSpec((_TC_R, D), lambda i: (i, 0)),
        out_shape=jax.ShapeDtypeStruct((N, D), jnp.float32),
    )(h, agg, agg, degp, degp, Ws, Wn, b)


def kernel(features, edge_index, W_self0, W_neigh0, b0, W_self1, W_neigh1, b1):
    src = edge_index[0]
    dst = edge_index[1]
    pad = E_PAD - E
    ar = jnp.arange(pad, dtype=jnp.int32)
    src_p = jnp.concatenate([src, (ar * 37) % N]).reshape(NW * CPW, CHUNK)
    dst_p = jnp.concatenate([dst, N + ar % (ROWS_PAD - N)]).reshape(
        NW * CPW, CHUNK)
    z128 = jnp.zeros((RPS, D), jnp.float32)
    ones128 = jnp.ones((CHUNK_D, D), jnp.float32)
    dst2_p = dst_p.reshape(NW * CPW_D, CHUNK_D)

    degp = _sc_deg(dst2_p, z128, ones128)
    aggA = _sc_agg(features, src_p, dst_p, z128)
    h1 = _tc_layer(True, features, aggA, degp, W_self0, W_neigh0,
                   b0.reshape(1, D))
    aggB = _sc_agg(h1, src_p, dst_p, z128)
    out = _tc_layer(False, h1, aggB, degp, W_self1, W_neigh1,
                    b1.reshape(1, D))
    return out
